# baseline (speedup 1.0000x reference)
import jax
import jax.numpy as jnp
from jax import lax
from jax.experimental import pallas as pl

B, H, S, D, DM = 8, 16, 577, 1024, 64
DOM = 54
CTX = 10
KEEP = S - (DOM + 1)
STEP = max(1, KEEP // CTX)
NSEL = DOM + 1
OUT_T = NSEL + CTX
CROWS = 80

_HIGH = lax.Precision.HIGHEST
_DEF = lax.Precision.DEFAULT


def _mm(a, bm, prec=_DEF):
    return lax.dot_general(a, bm, (((1,), (0,)), ((), ())), precision=prec)


def _mm_nt(a, bm, prec=_DEF):
    return lax.dot_general(a, bm, (((1,), (1,)), ((), ())), precision=prec)


def _tn(x, ident, prec=_DEF):
    return lax.dot_general(x, ident, (((0,), (0,)), ((), ())),
                           precision=prec)


def _kernel(attn_cls_ref, hid_ref, met_ref, out_ref, idx_ref):
    f32 = jnp.float32
    attn = attn_cls_ref[0]
    v = jnp.sum(attn, axis=0, keepdims=True)
    col0 = lax.broadcasted_iota(jnp.int32, (1, S), 1) == 0
    v = jnp.where(col0, -jnp.inf, v)

    ident_s = (lax.broadcasted_iota(jnp.int32, (S, S), 0)
               == lax.broadcasted_iota(jnp.int32, (S, S), 1)).astype(f32)
    ii = lax.broadcasted_iota(jnp.int32, (S, S), 0)
    jj = lax.broadcasted_iota(jnp.int32, (S, S), 1)

    vcol = _tn(v, jnp.ones((1, 1), f32), _HIGH)
    vi = jnp.broadcast_to(vcol, (S, S))
    vj = jnp.broadcast_to(v, (S, S))
    beats = (vj > vi) | ((vj == vi) & (jj < ii))
    rank = jnp.sum(beats.astype(f32), axis=1, keepdims=True)
    selc = rank < float(DOM)

    pp = lax.broadcasted_iota(jnp.int32, (S, 128), 1).astype(f32)
    in_slot = ((jnp.broadcast_to(rank, (S, 128)) == pp - 1.0)
               & jnp.broadcast_to(selc, (S, 128)))
    ii128 = lax.broadcasted_iota(jnp.int32, (S, 128), 0)
    idx_row = jnp.sum(jnp.where(in_slot, ii128, 0), axis=0,
                      keepdims=True)
    idx_ref[0, 0:1, :] = idx_row
    tok = lax.broadcasted_iota(jnp.int32, (1, S), 1).astype(f32)

    iota_col0 = lax.broadcasted_iota(jnp.int32, (S, 1), 0) == 0
    sel_col = (selc | iota_col0).astype(f32)
    sel_row = _tn(sel_col, ident_s)
    lt = (ii < jj).astype(f32)
    dom_rank = _mm(sel_row, lt)
    unsel_row = 1.0 - sel_row
    f_rank = tok - dom_rank

    t_id = jnp.floor((f_rank + 0.5) * (1.0 / STEP))
    is_target = ((unsel_row > 0.5) & (f_rank == t_id * float(STEP))
                 & (f_rank < float(STEP * CTX)))
    is_merge = (unsel_row > 0.5) & (~is_target)

    met = met_ref[0]
    ss = jnp.sum(met * met, axis=1, keepdims=True)
    mn = met / jnp.sqrt(ss)
    tt16 = lax.broadcasted_iota(jnp.int32, (16, S), 0).astype(f32)
    wt = ((jnp.broadcast_to(t_id, (16, S)) == tt16)
          & jnp.broadcast_to(is_target, (16, S))).astype(f32)
    tmat = _mm(wt, mn, _HIGH)
    sim = _mm_nt(mn, tmat)
    tcol = lax.broadcasted_iota(jnp.int32, (S, 16), 1).astype(f32)
    sim = jnp.where(tcol < float(CTX), sim, -jnp.inf)
    mx = jnp.max(sim, axis=1, keepdims=True)
    assign_col = jnp.min(jnp.where(sim == mx, tcol, 1e9), axis=1,
                         keepdims=True)
    amat = (jnp.broadcast_to(assign_col, (S, 16)) == tcol).astype(f32)
    counts = _mm(is_merge.astype(f32), amat)
    inv_counts = 1.0 / jnp.maximum(counts, 1.0)
    invc_row = _mm_nt(inv_counts, amat, _HIGH)
    assign_row = _tn(assign_col, ident_s)

    rr = lax.broadcasted_iota(jnp.int32, (CROWS, S), 0).astype(f32)
    dom_part = ((jnp.broadcast_to(dom_rank, (CROWS, S)) == rr)
                & jnp.broadcast_to(sel_row > 0.5, (CROWS, S)))
    tgt_part = ((jnp.broadcast_to(t_id, (CROWS, S)) == rr - 64.0)
                & jnp.broadcast_to(is_target, (CROWS, S)))
    mrg_part = ((jnp.broadcast_to(assign_row, (CROWS, S)) == rr - 64.0)
                & jnp.broadcast_to(is_merge, (CROWS, S)))
    c = (dom_part.astype(f32) + tgt_part.astype(f32)
         + mrg_part.astype(f32) * jnp.broadcast_to(invc_row, (CROWS, S)))

    hid = hid_ref[0]
    out = _mm(c, hid, _HIGH)
    out_ref[0, 0:NSEL, :] = out[0:NSEL, :]
    out_ref[0, NSEL:OUT_T, :] = out[64:64 + CTX, :]


@jax.jit
def kernel(attn_weights, hidden_states, metric):
    attn_cls = attn_weights[:, :, 0, :]
    out, idx = pl.pallas_call(
        _kernel,
        grid=(B,),
        in_specs=[
            pl.BlockSpec((1, H, S), lambda b: (b, 0, 0)),
            pl.BlockSpec((1, S, D), lambda b: (b, 0, 0)),
            pl.BlockSpec((1, S, DM), lambda b: (b, 0, 0)),
        ],
        out_specs=[
            pl.BlockSpec((1, OUT_T, D), lambda b: (b, 0, 0)),
            pl.BlockSpec((1, 8, 128), lambda b: (b, 0, 0)),
        ],
        out_shape=[
            jax.ShapeDtypeStruct((B, OUT_T, D), jnp.float32),
            jax.ShapeDtypeStruct((B, 8, 128), jnp.int32),
        ],
    )(attn_cls, hidden_states, metric)
    return out, idx[:, 0, :NSEL]


# speedup vs baseline: 1.1027x; 1.1027x over previous
import jax
import jax.numpy as jnp
from jax import lax
from jax.experimental import pallas as pl

B, H, S, D, DM = 8, 16, 577, 1024, 64
DOM = 54
CTX = 10
KEEP = S - (DOM + 1)
STEP = max(1, KEEP // CTX)
NSEL = DOM + 1
OUT_T = NSEL + CTX
GCOLS = 56

_HIGH = lax.Precision.HIGHEST
_DEF = lax.Precision.DEFAULT


def _mm_tn(a, bm, prec=_DEF):
    return lax.dot_general(a, bm, (((0,), (0,)), ((), ())), precision=prec)


def _mm_nt(a, bm, prec=_DEF):
    return lax.dot_general(a, bm, (((1,), (1,)), ((), ())), precision=prec)


def _mm(a, bm, prec=_DEF):
    return lax.dot_general(a, bm, (((1,), (0,)), ((), ())), precision=prec)


def _kernel(attn_cls_ref, hid_ref, met_ref, out_ref, idx_ref):
    f32 = jnp.float32
    attn = attn_cls_ref[0]
    v = jnp.sum(attn, axis=0, keepdims=True)
    col0 = lax.broadcasted_iota(jnp.int32, (1, S), 1) == 0
    v = jnp.where(col0, -jnp.inf, v)
    vcol = _mm_tn(v, jnp.ones((1, 1), f32), _HIGH)

    ii = lax.broadcasted_iota(jnp.int32, (S, S), 0)
    jj = lax.broadcasted_iota(jnp.int32, (S, S), 1)

    vi = jnp.broadcast_to(vcol, (S, S))
    vj = jnp.broadcast_to(v, (S, S))
    beats = (vi > vj) | ((vi == vj) & (ii < jj))
    nbeat = jnp.sum(beats.astype(jnp.int32), axis=1, keepdims=True)
    rank = (S - 1) - nbeat
    selc = rank < DOM

    pp = lax.broadcasted_iota(jnp.int32, (S, 128), 1)
    in_slot = ((jnp.broadcast_to(rank, (S, 128)) == pp - 1)
               & jnp.broadcast_to(selc, (S, 128)))
    ii128 = lax.broadcasted_iota(jnp.int32, (S, 128), 0)
    idx_row = jnp.sum(jnp.where(in_slot, ii128, 0), axis=0,
                      keepdims=True)
    idx_ref[0, 0:1, :] = idx_row

    icol = lax.broadcasted_iota(jnp.int32, (S, 1), 0)
    sel_col = (selc | (icol == 0)).astype(f32)
    ltT = (jj < ii).astype(f32)
    dom_rank = _mm(ltT, sel_col)
    unsel = 1.0 - sel_col
    f_rank = icol.astype(f32) - dom_rank

    t_id = jnp.floor((f_rank + 0.5) * (1.0 / STEP))
    is_target = ((unsel > 0.5) & (f_rank == t_id * float(STEP))
                 & (f_rank < float(STEP * CTX)))
    is_merge = (unsel > 0.5) & (~is_target)

    met = met_ref[0]
    ss = jnp.sum(met * met, axis=1, keepdims=True)
    mn = met / jnp.sqrt(ss)
    trow = lax.broadcasted_iota(jnp.int32, (S, 16), 1).astype(f32)
    tsel = ((jnp.broadcast_to(t_id, (S, 16)) == trow)
            & jnp.broadcast_to(is_target, (S, 16))).astype(f32)
    tmat = _mm_tn(tsel, mn, _HIGH)
    sim = _mm_nt(mn, tmat)
    sim = jnp.where(trow < float(CTX), sim, -jnp.inf)
    mx = jnp.max(sim, axis=1, keepdims=True)
    assign = jnp.min(jnp.where(sim == mx, trow, 1e9), axis=1,
                     keepdims=True)
    amat = ((jnp.broadcast_to(assign, (S, 16)) == trow)
            & jnp.broadcast_to(is_merge, (S, 16))).astype(f32)
    counts = jnp.sum(amat, axis=0, keepdims=True)
    inv_counts = 1.0 / jnp.maximum(counts, 1.0)
    ct = tsel + amat * jnp.broadcast_to(inv_counts, (S, 16))

    gr = lax.broadcasted_iota(jnp.int32, (S, GCOLS), 1).astype(f32)
    gt = ((jnp.broadcast_to(dom_rank, (S, GCOLS)) == gr)
          & jnp.broadcast_to(sel_col > 0.5, (S, GCOLS))).astype(f32)

    hid = hid_ref[0]
    out_dom = _mm_tn(gt, hid, _HIGH)
    out_ctx = _mm_tn(ct, hid)
    out_ref[0, 0:NSEL, :] = out_dom[0:NSEL, :]
    out_ref[0, NSEL:OUT_T, :] = out_ctx[0:CTX, :]


@jax.jit
def kernel(attn_weights, hidden_states, metric):
    attn_cls = attn_weights[:, :, 0, :]
    out, idx = pl.pallas_call(
        _kernel,
        grid=(B,),
        in_specs=[
            pl.BlockSpec((1, H, S), lambda b: (b, 0, 0)),
            pl.BlockSpec((1, S, D), lambda b: (b, 0, 0)),
            pl.BlockSpec((1, S, DM), lambda b: (b, 0, 0)),
        ],
        out_specs=[
            pl.BlockSpec((1, OUT_T, D), lambda b: (b, 0, 0)),
            pl.BlockSpec((1, 8, 128), lambda b: (b, 0, 0)),
        ],
        out_shape=[
            jax.ShapeDtypeStruct((B, OUT_T, D), jnp.float32),
            jax.ShapeDtypeStruct((B, 8, 128), jnp.int32),
        ],
    )(attn_cls, hidden_states, metric)
    return out, idx[:, 0, :NSEL]


# speedup vs baseline: 1.4723x; 1.3351x over previous
import jax
import jax.numpy as jnp
from jax import lax
from jax.experimental import pallas as pl

B, H, S, D, DM = 8, 16, 577, 1024, 64
DOM = 54
CTX = 10
KEEP = S - (DOM + 1)
STEP = max(1, KEEP // CTX)
NSEL = DOM + 1
OUT_T = NSEL + CTX
GCOLS = 56

_HIGH = lax.Precision.HIGHEST
_DEF = lax.Precision.DEFAULT


def _mm_tn(a, bm, prec=_DEF):
    return lax.dot_general(a, bm, (((0,), (0,)), ((), ())), precision=prec)


def _mm(a, bm, prec=_DEF):
    return lax.dot_general(a, bm, (((1,), (0,)), ((), ())), precision=prec)


def _kernel(attn_cls_ref, hid_ref, met_ref, out_ref, idx_ref):
    f32 = jnp.float32
    b = pl.program_id(0)

    ii = lax.broadcasted_iota(jnp.int32, (S, S), 0)
    jj = lax.broadcasted_iota(jnp.int32, (S, S), 1)

    attn = attn_cls_ref[b]
    v = jnp.sum(attn, axis=0, keepdims=True)
    col0 = lax.broadcasted_iota(jnp.int32, (1, S), 1) == 0
    v = jnp.where(col0, -jnp.inf, v)
    vcol = _mm_tn(v, jnp.ones((1, 1), f32), _HIGH)

    vi = jnp.broadcast_to(vcol, (S, S))
    vj = jnp.broadcast_to(v, (S, S))
    beats = (vi > vj) | ((vi == vj) & (ii < jj))
    nbeat = jnp.sum(beats.astype(jnp.int32), axis=1, keepdims=True)
    rank = (S - 1) - nbeat
    selc = rank < DOM

    pp = lax.broadcasted_iota(jnp.int32, (S, 128), 1)
    in_slot = ((jnp.broadcast_to(rank, (S, 128)) == pp - 1)
               & jnp.broadcast_to(selc, (S, 128)))
    ii128 = lax.broadcasted_iota(jnp.int32, (S, 128), 0)
    idx_row = jnp.sum(jnp.where(in_slot, ii128, 0), axis=0,
                      keepdims=True)
    idx_ref[b, 0:1, :] = idx_row

    icol = lax.broadcasted_iota(jnp.int32, (S, 1), 0)
    sel_col = (selc | (icol == 0)).astype(f32)
    ltT = (jj < ii).astype(f32)
    dom_rank = _mm(ltT, sel_col)
    unsel = 1.0 - sel_col
    f_rank = icol.astype(f32) - dom_rank

    t_id = jnp.floor((f_rank + 0.5) * (1.0 / STEP))
    is_target = ((unsel > 0.5) & (f_rank == t_id * float(STEP))
                 & (f_rank < float(STEP * CTX)))
    is_merge = (unsel > 0.5) & (~is_target)

    mt = met_ref[b]
    ss = jnp.sum(mt * mt, axis=0, keepdims=True)
    mn_t = mt / jnp.sqrt(ss)
    trow = lax.broadcasted_iota(jnp.int32, (S, 16), 1).astype(f32)
    tsel = ((jnp.broadcast_to(t_id, (S, 16)) == trow)
            & jnp.broadcast_to(is_target, (S, 16))).astype(f32)
    tmat_t = _mm(mn_t, tsel, _HIGH)
    sim = _mm_tn(mn_t, tmat_t)
    sim = jnp.where(trow < float(CTX), sim, -jnp.inf)
    mx = jnp.max(sim, axis=1, keepdims=True)
    assign = jnp.min(jnp.where(sim == mx, trow, 1e9), axis=1,
                     keepdims=True)
    amat = ((jnp.broadcast_to(assign, (S, 16)) == trow)
            & jnp.broadcast_to(is_merge, (S, 16))).astype(f32)
    counts = jnp.sum(amat, axis=0, keepdims=True)
    inv_counts = 1.0 / jnp.maximum(counts, 1.0)
    ct = tsel + amat * jnp.broadcast_to(inv_counts, (S, 16))

    gr = lax.broadcasted_iota(jnp.int32, (S, GCOLS), 1).astype(f32)
    gt = ((jnp.broadcast_to(dom_rank, (S, GCOLS)) == gr)
          & jnp.broadcast_to(sel_col > 0.5, (S, GCOLS))).astype(f32)

    hid = hid_ref[:, b, :]
    out_dom = _mm_tn(gt, hid, _HIGH)
    out_ctx = _mm_tn(ct, hid)
    out_ref[0:NSEL, b, :] = out_dom[0:NSEL, :]
    out_ref[NSEL:OUT_T, b, :] = out_ctx[0:CTX, :]


@jax.jit
def kernel(attn_weights, hidden_states, metric):
    attn_cls = attn_weights[:, :, 0, :]
    hid_t = jnp.transpose(hidden_states, (1, 0, 2))
    met_t = jnp.transpose(metric, (0, 2, 1))
    out_t, idx = pl.pallas_call(
        _kernel,
        grid=(B,),
        in_specs=[
            pl.BlockSpec((B, H, S), lambda b: (0, 0, 0)),
            pl.BlockSpec((S, B, D), lambda b: (0, 0, 0)),
            pl.BlockSpec((B, DM, S), lambda b: (0, 0, 0)),
        ],
        out_specs=[
            pl.BlockSpec((OUT_T, B, D), lambda b: (0, 0, 0)),
            pl.BlockSpec((B, 8, 128), lambda b: (0, 0, 0)),
        ],
        out_shape=[
            jax.ShapeDtypeStruct((OUT_T, B, D), jnp.float32),
            jax.ShapeDtypeStruct((B, 8, 128), jnp.int32),
        ],
    )(attn_cls, hid_t, met_t)
    return jnp.transpose(out_t, (1, 0, 2)), idx[:, 0, :NSEL]
